# baseline (speedup 1.0000x reference)
import functools

import jax
import jax.numpy as jnp
from jax.experimental import pallas as pl
from jax.experimental.pallas import tpu as pltpu

HIDDEN = 1024
FFN = 1024
NUM_EXPERTS = 64
TOP_K = 2
TOKENS = 128


def _router_kernel(x_ref, gw_ref, logits_ref, comb_ref, mask_ref):
    logits = jax.lax.dot_general(
        x_ref[...], gw_ref[...], (((1,), (0,)), ((), ())),
        preferred_element_type=jnp.float32,
    )
    logits_ref[...] = logits
    m = jnp.max(logits, axis=-1, keepdims=True)
    ex = jnp.exp(logits - m)
    probs = ex / jnp.sum(ex, axis=-1, keepdims=True)
    col = jax.lax.broadcasted_iota(jnp.int32, probs.shape, 1)
    big = jnp.int32(NUM_EXPERTS)
    m1 = jnp.max(probs, axis=-1, keepdims=True)
    i1 = jnp.min(jnp.where(probs == m1, col, big), axis=-1, keepdims=True)
    oh1 = col == i1
    probs2 = jnp.where(oh1, -1.0, probs)
    m2 = jnp.max(probs2, axis=-1, keepdims=True)
    i2 = jnp.min(jnp.where(probs2 == m2, col, big), axis=-1, keepdims=True)
    oh2 = col == i2
    denom = m1 + m2
    comb = (jnp.where(oh1, m1, 0.0) + jnp.where(oh2, m2, 0.0)) / denom
    comb_ref[...] = comb
    sel = jnp.logical_or(oh1, oh2)
    mask_ref[...] = jnp.max(sel.astype(jnp.int32), axis=0, keepdims=True)


def _expert_kernel(perm_ref, cnt_ref, x_ref, w1_ref, w2_ref, comb_ref, out_ref):
    i = pl.program_id(0)

    @pl.when(i == 0)
    def _():
        out_ref[...] = jnp.zeros_like(out_ref)

    @pl.when(i < cnt_ref[0])
    def _():
        e = perm_ref[i]
        x = x_ref[...].astype(jnp.bfloat16)
        w1e = w1_ref[0].astype(jnp.bfloat16)
        h = jax.lax.dot_general(
            x, w1e, (((1,), (1,)), ((), ())), preferred_element_type=jnp.float32
        )
        gate = h[:, :FFN]
        up = h[:, FFN:]
        act = gate * jax.lax.logistic(gate) * up
        w2e = w2_ref[0].astype(jnp.bfloat16)
        eo = jax.lax.dot_general(
            act.astype(jnp.bfloat16), w2e, (((1,), (1,)), ((), ())),
            preferred_element_type=jnp.float32,
        )
        comb = comb_ref[...]
        col = jax.lax.broadcasted_iota(jnp.int32, comb.shape, 1)
        cw = jnp.sum(jnp.where(col == e, comb, 0.0), axis=1, keepdims=True)
        out_ref[...] += cw * eo


def _clamped(i, perm_ref, cnt_ref):
    return perm_ref[jnp.minimum(i, cnt_ref[0] - 1)]


@jax.jit
def kernel(hidden_states, gate_w, w1, w2):
    logits, comb, mask2d = pl.pallas_call(
        _router_kernel,
        out_shape=(
            jax.ShapeDtypeStruct((TOKENS, NUM_EXPERTS), jnp.float32),
            jax.ShapeDtypeStruct((TOKENS, NUM_EXPERTS), jnp.float32),
            jax.ShapeDtypeStruct((1, NUM_EXPERTS), jnp.int32),
        ),
    )(hidden_states, gate_w)

    mask = mask2d.reshape(NUM_EXPERTS)
    perm = jnp.argsort(1 - mask, stable=True).astype(jnp.int32)
    cnt = jnp.sum(mask, dtype=jnp.int32).reshape(1)

    grid_spec = pltpu.PrefetchScalarGridSpec(
        num_scalar_prefetch=2,
        grid=(NUM_EXPERTS,),
        in_specs=[
            pl.BlockSpec((TOKENS, HIDDEN), lambda i, p, c: (0, 0)),
            pl.BlockSpec(
                (1, 2 * FFN, HIDDEN),
                lambda i, p, c: (_clamped(i, p, c), 0, 0),
            ),
            pl.BlockSpec(
                (1, HIDDEN, FFN),
                lambda i, p, c: (_clamped(i, p, c), 0, 0),
            ),
            pl.BlockSpec((TOKENS, NUM_EXPERTS), lambda i, p, c: (0, 0)),
        ],
        out_specs=pl.BlockSpec((TOKENS, HIDDEN), lambda i, p, c: (0, 0)),
    )
    out = pl.pallas_call(
        _expert_kernel,
        grid_spec=grid_spec,
        out_shape=jax.ShapeDtypeStruct((TOKENS, HIDDEN), jnp.float32),
    )(perm, cnt, hidden_states, w1, w2, comb)

    return (out, logits)
